# baseline (speedup 1.0000x reference)
import functools

import jax
import jax.numpy as jnp
from jax import lax
from jax.experimental import pallas as pl
from jax.experimental.pallas import tpu as pltpu
from jax.experimental.pallas import tpu_sc as plsc

D = 128
H = 8
DH = 16
LANES = 16
NUM_CORES = 2
NUM_SUBCORES = 16

CHUNK = 40
ACCW = 144
ROW_PAD = NUM_SUBCORES * 128


def _proj_body(h_ref, sv_ref, wq_sv, wk_sv, wv_sv, wq_cv, wk_cv, wv_cv,
               wq_sh, wk_sh, wv_sh,
               kv0, q0, kv1, q1, kv2, q2, kv3, q3):
    hb = h_ref[...]
    svb = sv_ref[...]

    def mm(a, w):
        return jnp.dot(a, w[...], preferred_element_type=jnp.float32)

    kv0[:, :D] = mm(hb, wk_sv)
    kv0[:, D:] = mm(hb, wv_sv)
    q0[...] = mm(hb, wq_sv)
    kv1[:, :D] = mm(svb, wk_cv)
    kv1[:, D:] = mm(svb, wv_cv)
    q1[...] = mm(hb, wq_cv)
    kv2[:, :D] = mm(svb, wk_sh)
    kv2[:, D:] = mm(svb, wv_sh)
    q2[...] = mm(svb, wq_sh)
    kv3[:, :D] = mm(hb, wk_cv)
    kv3[:, D:] = mm(hb, wv_cv)
    q3[...] = mm(svb, wq_cv)


def _projections(h, sv, p):
    n = h.shape[0]
    blk = 2000
    grid = n // blk
    row_spec = pl.BlockSpec((blk, D), lambda i: (i, 0))
    w_spec = pl.BlockSpec((D, D), lambda i: (0, 0))
    kv_spec = pl.BlockSpec((blk, 2 * D), lambda i: (i, 0))
    return pl.pallas_call(
        _proj_body,
        grid=(grid,),
        in_specs=[row_spec, row_spec] + [w_spec] * 9,
        out_specs=[kv_spec, row_spec] * 4,
        out_shape=[
            s for _ in range(4)
            for s in (jax.ShapeDtypeStruct((n, 2 * D), jnp.float32),
                      jax.ShapeDtypeStruct((n, D), jnp.float32))
        ],
    )(h, sv, p['Wq_sv'], p['Wk_sv'], p['Wv_sv'], p['Wq_cv'], p['Wk_cv'],
      p['Wv_cv'], p['Wq_sh'], p['Wk_sh'], p['Wv_sh'])


def _make_sc_edge_kernel(npad, e):
    ept = e // NUM_SUBCORES
    n_chunks = ept // CHUNK
    rows_per_tile = npad // NUM_SUBCORES
    nz = rows_per_tile // CHUNK

    mesh = plsc.VectorSubcoreMesh(core_axis_name="c", subcore_axis_name="s",
                                  num_cores=NUM_CORES,
                                  num_subcores=NUM_SUBCORES)

    @functools.partial(
        pl.kernel,
        mesh=mesh,
        compiler_params=pltpu.CompilerParams(needs_layout_passes=False,
                                             use_tc_tiling_on_sc=False),
        out_type=[jax.ShapeDtypeStruct((npad, ACCW), jnp.float32)] * 4,
        scratch_types=[
            pltpu.VMEM_SHARED((npad, ACCW), jnp.float32),
            pltpu.VMEM((CHUNK,), jnp.int32),
            pltpu.VMEM((CHUNK,), jnp.int32),
            pltpu.VMEM((CHUNK, 2 * D), jnp.float32),
            pltpu.VMEM((CHUNK, D), jnp.float32),
            pltpu.VMEM((CHUNK, ACCW), jnp.float32),
            pltpu.SemaphoreType.DMA,
            pltpu.SemaphoreType.DMA,
        ],
    )
    def sc_kernel(kv0, q0, kv1, q1, kv2, q2, kv3, q3, src_hbm, dst_hbm,
                  out0, out1, out2, out3,
                  acc, srcb, dstb, kvb, qb, msgb, sem_kv, sem_q):
        cid = lax.axis_index("c")
        sid = lax.axis_index("s")
        zero16 = jnp.zeros((LANES,), jnp.float32)
        iot = lax.broadcasted_iota(jnp.int32, (LANES,), 0)

        def run_attention(kv_hbm, q_hbm, out_hbm):
            def zero_row(r, carry):
                for kk in range(ACCW // LANES):
                    msgb[r, pl.ds(kk * LANES, LANES)] = zero16
                return carry

            lax.fori_loop(0, CHUNK, zero_row, 0)
            for z in range(nz):
                pltpu.sync_copy(
                    msgb,
                    acc.at[pl.ds(sid * rows_per_tile + z * CHUNK, CHUNK)])
            plsc.subcore_barrier()

            def chunk_body(i, carry):
                base = sid * ept + i * CHUNK
                pltpu.sync_copy(src_hbm.at[pl.ds(base, CHUNK)], srcb)
                pltpu.sync_copy(dst_hbm.at[pl.ds(base, CHUNK)], dstb)
                cp_kv = pltpu.async_copy(kv_hbm.at[srcb], kvb, sem_kv)
                cp_q = pltpu.async_copy(q_hbm.at[dstb], qb, sem_q)
                cp_kv.wait()
                cp_q.wait()

                @plsc.parallel_loop(0, CHUNK, unroll=8)
                def edge_body(ei):
                    zv = zero16
                    for hh in range(H):
                        kvec = kvb[ei, pl.ds(hh * DH, DH)]
                        qvec = qb[ei, pl.ds(hh * DH, DH)]
                        dsc = jnp.sum(kvec * qvec) * 0.25
                        dsc = jnp.minimum(jnp.maximum(dsc, -5.0), 5.0)
                        svec = jnp.exp(jnp.full((LANES,), dsc, jnp.float32))
                        vvec = kvb[ei, pl.ds(D + hh * DH, DH)]
                        msgb[ei, pl.ds(hh * DH, DH)] = svec * vvec
                        zv = jnp.where(iot == hh, svec, zv)
                    msgb[ei, pl.ds(D, LANES)] = zv
                pltpu.sync_copy(msgb, acc.at[dstb], add=True)
                return carry

            lax.fori_loop(0, n_chunks, chunk_body, 0)
            plsc.subcore_barrier()
            pltpu.sync_copy(
                acc.at[pl.ds(sid * rows_per_tile, rows_per_tile)],
                out_hbm.at[pl.ds(sid * rows_per_tile, rows_per_tile)])
            plsc.subcore_barrier()

        @pl.when(cid == 0)
        def _():
            run_attention(kv0, q0, out0)
            run_attention(kv1, q1, out1)

        @pl.when(cid == 1)
        def _():
            run_attention(kv2, q2, out2)
            run_attention(kv3, q3, out3)

    return sc_kernel


def _mm(a, w):
    return jnp.dot(a, w, preferred_element_type=jnp.float32)


def _accum_stats(ref, x):
    st = jnp.concatenate([jnp.sum(x, axis=0, keepdims=True),
                          jnp.sum(x * x, axis=0, keepdims=True)], axis=0)

    @pl.when(pl.program_id(0) == 0)
    def _():
        ref[...] = st

    @pl.when(pl.program_id(0) != 0)
    def _():
        ref[...] += st


def _bn_apply(x, st_ref, n, g, b):
    st = st_ref[...]
    m = st[0:1, :] * (1.0 / n)
    v = st[1:2, :] * (1.0 / n) - m * m
    return g[...] * (x - m) * lax.rsqrt(v + 1e-5) + b[...]


def _post1_body(acc0, acc1, acc2, acc3, h_ref, sv_ref,
                o_w, o_b, oh_w, oh_b, g1_w, g1_b,
                x1_out, s1_out, stx_ref, sts_ref):
    f32 = jnp.float32
    rowi = lax.broadcasted_iota(jnp.int32, (H, D), 0)
    coli = lax.broadcasted_iota(jnp.int32, (H, D), 1)
    expand = (coli // DH == rowi).astype(f32)

    def att(acc_ref):
        a = acc_ref[...]
        return a[:, :D] / _mm(a[:, D:D + H], expand)

    hb = h_ref[...]
    svb = sv_ref[...]
    ow = o_w[...]
    x1 = _mm(att(acc0), ow[:D, :]) + _mm(att(acc1), ow[D:, :]) + o_b[...]
    x1 = hb + x1
    x1_out[...] = x1
    _accum_stats(stx_ref, x1)

    ohw = oh_w[...]
    s1 = _mm(att(acc2), ohw[:D, :]) + _mm(att(acc3), ohw[D:, :]) + oh_b[...]
    g1 = jax.nn.sigmoid(_mm(hb, g1_w[...]) + g1_b[...])
    s1 = (1.0 - g1) * svb + g1 * s1
    s1_out[...] = s1
    _accum_stats(sts_ref, s1)


def _post2_body(n, x1_ref, s1_ref, h_ref, stx_ref, sts_ref,
                f1_w, f1_b, f2_w, f2_b, bn1_g, bn1_b,
                f1h_w, f1h_b, f2h_w, f2h_b, bn1h_g, bn1h_b,
                g2_w, g2_b,
                x2_out, s2_out, stx2_ref, sts2_ref):
    def relu(x):
        return jnp.maximum(x, 0.0)

    xb1 = _bn_apply(x1_ref[...], stx_ref, n, bn1_g, bn1_b)
    y = _mm(relu(_mm(xb1, f1_w[...]) + f1_b[...]), f2_w[...]) + f2_b[...]
    x2 = xb1 + y
    x2_out[...] = x2
    _accum_stats(stx2_ref, x2)

    sb1 = _bn_apply(s1_ref[...], sts_ref, n, bn1h_g, bn1h_b)
    y2 = _mm(relu(_mm(sb1, f1h_w[...]) + f1h_b[...]), f2h_w[...]) + f2h_b[...]
    g2 = jax.nn.sigmoid(_mm(h_ref[...], g2_w[...]) + g2_b[...])
    s2 = (1.0 - g2) * sb1 + g2 * y2
    s2_out[...] = s2
    _accum_stats(sts2_ref, s2)


def _post3_body(n, x2_ref, s2_ref, stx2_ref, sts2_ref,
                bn2_g, bn2_b, bn2h_g, bn2h_b, x_out, s_out):
    x_out[...] = _bn_apply(x2_ref[...], stx2_ref, n, bn2_g, bn2_b)
    s_out[...] = _bn_apply(s2_ref[...], sts2_ref, n, bn2h_g, bn2h_b)


def _post(accs, h, sv, p):
    n = h.shape[0]
    blk = 2000
    grid = n // blk

    def v2(a):
        return a.reshape(1, -1)

    row = pl.BlockSpec((blk, D), lambda i: (i, 0))
    accs_spec = pl.BlockSpec((blk, ACCW), lambda i: (i, 0))
    st = pl.BlockSpec((2, D), lambda i: (0, 0))

    def wspec(a):
        return pl.BlockSpec(a.shape, lambda i: (0,) * a.ndim)

    nd = jax.ShapeDtypeStruct((n, D), jnp.float32)
    std = jax.ShapeDtypeStruct((2, D), jnp.float32)

    w1 = (p['O_w'], v2(p['O_b']), p['Oh_w'], v2(p['Oh_b']),
          p['G1_w'], v2(p['G1_b']))
    x1, s1, stx, sts = pl.pallas_call(
        _post1_body,
        grid=(grid,),
        in_specs=[accs_spec] * 4 + [row, row] + [wspec(a) for a in w1],
        out_specs=[row, row, st, st],
        out_shape=[nd, nd, std, std],
    )(*accs, h, sv, *w1)

    w2 = (p['F1_w'], v2(p['F1_b']), p['F2_w'], v2(p['F2_b']),
          v2(p['bn1_g']), v2(p['bn1_b']),
          p['F1h_w'], v2(p['F1h_b']), p['F2h_w'], v2(p['F2h_b']),
          v2(p['bn1h_g']), v2(p['bn1h_b']), p['G2_w'], v2(p['G2_b']))
    x2, s2m, stx2, sts2 = pl.pallas_call(
        functools.partial(_post2_body, n),
        grid=(grid,),
        in_specs=[row, row, row, st, st] + [wspec(a) for a in w2],
        out_specs=[row, row, st, st],
        out_shape=[nd, nd, std, std],
    )(x1, s1, h, stx, sts, *w2)

    w3 = (v2(p['bn2_g']), v2(p['bn2_b']), v2(p['bn2h_g']), v2(p['bn2h_b']))
    x, s2 = pl.pallas_call(
        functools.partial(_post3_body, n),
        grid=(grid,),
        in_specs=[row, row, st, st] + [wspec(a) for a in w3],
        out_specs=[row, row],
        out_shape=[nd, nd],
    )(x2, s2m, stx2, sts2, *w3)
    return x, s2


def kernel(h, state_vectors, edge_index, params):
    n = h.shape[0]
    e = edge_index.shape[1]
    src = edge_index[0]
    dst = edge_index[1]

    npad = -(-n // ROW_PAD) * ROW_PAD

    kv0, q0, kv1, q1, kv2, q2, kv3, q3 = _projections(h, state_vectors, params)
    accs = _make_sc_edge_kernel(npad, e)(
        kv0, q0, kv1, q1, kv2, q2, kv3, q3, src, dst)
    x, s2 = _post(accs, h, state_vectors, params)
    return x, s2


# speedup vs baseline: 4.3154x; 4.3154x over previous
import functools

import jax
import jax.numpy as jnp
from jax import lax
from jax.experimental import pallas as pl
from jax.experimental.pallas import tpu as pltpu
from jax.experimental.pallas import tpu_sc as plsc

D = 128
H = 8
DH = 16
LANES = 16
NUM_CORES = 2
NUM_SUBCORES = 16

CHUNK = 40
ACCW = 144
ROW_PAD = NUM_SUBCORES * 128


def _proj_body(h_ref, sv_ref, wq_sv, wk_sv, wv_sv, wq_cv, wk_cv, wv_cv,
               wq_sh, wk_sh, wv_sh,
               kv0, q0, kv1, q1, kv2, q2, kv3, q3):
    hb = h_ref[...]
    svb = sv_ref[...]

    def mm(a, w):
        return jnp.dot(a, w[...], preferred_element_type=jnp.float32)

    kv0[:, :D] = mm(hb, wk_sv)
    kv0[:, D:] = mm(hb, wv_sv)
    q0[...] = mm(hb, wq_sv)
    kv1[:, :D] = mm(svb, wk_cv)
    kv1[:, D:] = mm(svb, wv_cv)
    q1[...] = mm(hb, wq_cv)
    kv2[:, :D] = mm(svb, wk_sh)
    kv2[:, D:] = mm(svb, wv_sh)
    q2[...] = mm(svb, wq_sh)
    kv3[:, :D] = mm(hb, wk_cv)
    kv3[:, D:] = mm(hb, wv_cv)
    q3[...] = mm(svb, wq_cv)


def _projections(h, sv, p):
    n = h.shape[0]
    blk = 2000
    grid = n // blk
    row_spec = pl.BlockSpec((blk, D), lambda i: (i, 0))
    w_spec = pl.BlockSpec((D, D), lambda i: (0, 0))
    kv_spec = pl.BlockSpec((blk, 2 * D), lambda i: (i, 0))
    return pl.pallas_call(
        _proj_body,
        grid=(grid,),
        in_specs=[row_spec, row_spec] + [w_spec] * 9,
        out_specs=[kv_spec, row_spec] * 4,
        out_shape=[
            s for _ in range(4)
            for s in (jax.ShapeDtypeStruct((n, 2 * D), jnp.float32),
                      jax.ShapeDtypeStruct((n, D), jnp.float32))
        ],
    )(h, sv, p['Wq_sv'], p['Wk_sv'], p['Wv_sv'], p['Wq_cv'], p['Wk_cv'],
      p['Wv_cv'], p['Wq_sh'], p['Wk_sh'], p['Wv_sh'])


def _make_sc_edge_kernel(npad, e):
    ept = e // NUM_SUBCORES
    n_chunks = ept // CHUNK
    rows_per_tile = npad // NUM_SUBCORES
    nz = rows_per_tile // CHUNK

    mesh = plsc.VectorSubcoreMesh(core_axis_name="c", subcore_axis_name="s",
                                  num_cores=NUM_CORES,
                                  num_subcores=NUM_SUBCORES)

    @functools.partial(
        pl.kernel,
        mesh=mesh,
        compiler_params=pltpu.CompilerParams(needs_layout_passes=False,
                                             use_tc_tiling_on_sc=False),
        out_type=[jax.ShapeDtypeStruct((npad, ACCW), jnp.float32)] * 4,
        scratch_types=[
            pltpu.VMEM_SHARED((npad, ACCW), jnp.float32),
            pltpu.VMEM((2, 2, CHUNK), jnp.int32),
            pltpu.VMEM((2, CHUNK, 2 * D), jnp.float32),
            pltpu.VMEM((2, CHUNK, D), jnp.float32),
            pltpu.VMEM((CHUNK, ACCW), jnp.float32),
            pltpu.SemaphoreType.DMA((2,)),
            pltpu.SemaphoreType.DMA((2,)),
        ],
    )
    def sc_kernel(kv0, q0, kv1, q1, kv2, q2, kv3, q3, edge_hbm,
                  out0, out1, out2, out3,
                  acc, sdb, kvb, qb, msgb, sem_kv, sem_q):
        cid = lax.axis_index("c")
        sid = lax.axis_index("s")
        zero16 = jnp.zeros((LANES,), jnp.float32)
        iot = lax.broadcasted_iota(jnp.int32, (LANES,), 0)

        def run_attention(kv_hbm, q_hbm, out_hbm):
            def zero_row(r, carry):
                for kk in range(ACCW // LANES):
                    msgb[r, pl.ds(kk * LANES, LANES)] = zero16
                return carry

            lax.fori_loop(0, CHUNK, zero_row, 0)
            for z in range(nz):
                pltpu.sync_copy(
                    msgb,
                    acc.at[pl.ds(sid * rows_per_tile + z * CHUNK, CHUNK)])
            plsc.subcore_barrier()

            def stage_chunk(i, b):
                pltpu.sync_copy(
                    edge_hbm.at[:, pl.ds(sid * ept + i * CHUNK, CHUNK)],
                    sdb.at[b])
                pltpu.async_copy(kv_hbm.at[sdb.at[b, 0]], kvb.at[b],
                                 sem_kv.at[b])
                pltpu.async_copy(q_hbm.at[sdb.at[b, 1]], qb.at[b],
                                 sem_q.at[b])

            def gathers_wait(b):
                pltpu.make_async_copy(
                    kv_hbm.at[sdb.at[b, 0]], kvb.at[b], sem_kv.at[b]).wait()
                pltpu.make_async_copy(
                    q_hbm.at[sdb.at[b, 1]], qb.at[b], sem_q.at[b]).wait()

            stage_chunk(0, 0)

            def compute_scatter(p):
                kvb_p = kvb.at[p]
                qb_p = qb.at[p]

                @plsc.parallel_loop(0, CHUNK, unroll=1)
                def edge_body(ei):
                    zv = zero16
                    for hh in range(H):
                        kvec = kvb_p[ei, pl.ds(hh * DH, DH)]
                        qvec = qb_p[ei, pl.ds(hh * DH, DH)]
                        dsc = jnp.sum(kvec * qvec) * 0.25
                        dsc = jnp.minimum(jnp.maximum(dsc, -5.0), 5.0)
                        svec = jnp.exp(jnp.full((LANES,), dsc, jnp.float32))
                        vvec = kvb_p[ei, pl.ds(D + hh * DH, DH)]
                        msgb[ei, pl.ds(hh * DH, DH)] = svec * vvec
                        zv = jnp.where(iot == hh, svec, zv)
                    msgb[ei, pl.ds(D, LANES)] = zv

                pltpu.sync_copy(msgb, acc.at[sdb.at[p, 1]], add=True)

            def chunk_body(i, carry):
                b = lax.rem(i, 2)

                @pl.when(i + 1 < n_chunks)
                def _():
                    stage_chunk(i + 1, 1 - b)

                gathers_wait(b)

                @pl.when(b == 0)
                def _():
                    compute_scatter(0)

                @pl.when(b == 1)
                def _():
                    compute_scatter(1)

                return carry

            lax.fori_loop(0, n_chunks, chunk_body, 0)
            plsc.subcore_barrier()
            pltpu.sync_copy(
                acc.at[pl.ds(sid * rows_per_tile, rows_per_tile)],
                out_hbm.at[pl.ds(sid * rows_per_tile, rows_per_tile)])
            plsc.subcore_barrier()

        @pl.when(cid == 0)
        def _():
            run_attention(kv0, q0, out0)
            run_attention(kv1, q1, out1)

        @pl.when(cid == 1)
        def _():
            run_attention(kv2, q2, out2)
            run_attention(kv3, q3, out3)

    return sc_kernel


def _mm(a, w):
    return jnp.dot(a, w, preferred_element_type=jnp.float32)


def _accum_stats(ref, x):
    st = jnp.concatenate([jnp.sum(x, axis=0, keepdims=True),
                          jnp.sum(x * x, axis=0, keepdims=True)], axis=0)

    @pl.when(pl.program_id(0) == 0)
    def _():
        ref[...] = st

    @pl.when(pl.program_id(0) != 0)
    def _():
        ref[...] += st


def _bn_apply(x, st_ref, n, g, b):
    st = st_ref[...]
    m = st[0:1, :] * (1.0 / n)
    v = st[1:2, :] * (1.0 / n) - m * m
    return g[...] * (x - m) * lax.rsqrt(v + 1e-5) + b[...]


def _post1_body(acc0, acc1, acc2, acc3, h_ref, sv_ref,
                o_w, o_b, oh_w, oh_b, g1_w, g1_b,
                x1_out, s1_out, stx_ref, sts_ref):
    f32 = jnp.float32
    rowi = lax.broadcasted_iota(jnp.int32, (H, D), 0)
    coli = lax.broadcasted_iota(jnp.int32, (H, D), 1)
    expand = (coli // DH == rowi).astype(f32)

    def att(acc_ref):
        a = acc_ref[...]
        return a[:, :D] / _mm(a[:, D:D + H], expand)

    hb = h_ref[...]
    svb = sv_ref[...]
    ow = o_w[...]
    x1 = _mm(att(acc0), ow[:D, :]) + _mm(att(acc1), ow[D:, :]) + o_b[...]
    x1 = hb + x1
    x1_out[...] = x1
    _accum_stats(stx_ref, x1)

    ohw = oh_w[...]
    s1 = _mm(att(acc2), ohw[:D, :]) + _mm(att(acc3), ohw[D:, :]) + oh_b[...]
    g1 = jax.nn.sigmoid(_mm(hb, g1_w[...]) + g1_b[...])
    s1 = (1.0 - g1) * svb + g1 * s1
    s1_out[...] = s1
    _accum_stats(sts_ref, s1)


def _post2_body(n, x1_ref, s1_ref, h_ref, stx_ref, sts_ref,
                f1_w, f1_b, f2_w, f2_b, bn1_g, bn1_b,
                f1h_w, f1h_b, f2h_w, f2h_b, bn1h_g, bn1h_b,
                g2_w, g2_b,
                x2_out, s2_out, stx2_ref, sts2_ref):
    def relu(x):
        return jnp.maximum(x, 0.0)

    xb1 = _bn_apply(x1_ref[...], stx_ref, n, bn1_g, bn1_b)
    y = _mm(relu(_mm(xb1, f1_w[...]) + f1_b[...]), f2_w[...]) + f2_b[...]
    x2 = xb1 + y
    x2_out[...] = x2
    _accum_stats(stx2_ref, x2)

    sb1 = _bn_apply(s1_ref[...], sts_ref, n, bn1h_g, bn1h_b)
    y2 = _mm(relu(_mm(sb1, f1h_w[...]) + f1h_b[...]), f2h_w[...]) + f2h_b[...]
    g2 = jax.nn.sigmoid(_mm(h_ref[...], g2_w[...]) + g2_b[...])
    s2 = (1.0 - g2) * sb1 + g2 * y2
    s2_out[...] = s2
    _accum_stats(sts2_ref, s2)


def _post3_body(n, x2_ref, s2_ref, stx2_ref, sts2_ref,
                bn2_g, bn2_b, bn2h_g, bn2h_b, x_out, s_out):
    x_out[...] = _bn_apply(x2_ref[...], stx2_ref, n, bn2_g, bn2_b)
    s_out[...] = _bn_apply(s2_ref[...], sts2_ref, n, bn2h_g, bn2h_b)


def _post(accs, h, sv, p):
    n = h.shape[0]
    blk = 2000
    grid = n // blk

    def v2(a):
        return a.reshape(1, -1)

    row = pl.BlockSpec((blk, D), lambda i: (i, 0))
    accs_spec = pl.BlockSpec((blk, ACCW), lambda i: (i, 0))
    st = pl.BlockSpec((2, D), lambda i: (0, 0))

    def wspec(a):
        return pl.BlockSpec(a.shape, lambda i: (0,) * a.ndim)

    nd = jax.ShapeDtypeStruct((n, D), jnp.float32)
    std = jax.ShapeDtypeStruct((2, D), jnp.float32)

    w1 = (p['O_w'], v2(p['O_b']), p['Oh_w'], v2(p['Oh_b']),
          p['G1_w'], v2(p['G1_b']))
    x1, s1, stx, sts = pl.pallas_call(
        _post1_body,
        grid=(grid,),
        in_specs=[accs_spec] * 4 + [row, row] + [wspec(a) for a in w1],
        out_specs=[row, row, st, st],
        out_shape=[nd, nd, std, std],
    )(*accs, h, sv, *w1)

    w2 = (p['F1_w'], v2(p['F1_b']), p['F2_w'], v2(p['F2_b']),
          v2(p['bn1_g']), v2(p['bn1_b']),
          p['F1h_w'], v2(p['F1h_b']), p['F2h_w'], v2(p['F2h_b']),
          v2(p['bn1h_g']), v2(p['bn1h_b']), p['G2_w'], v2(p['G2_b']))
    x2, s2m, stx2, sts2 = pl.pallas_call(
        functools.partial(_post2_body, n),
        grid=(grid,),
        in_specs=[row, row, row, st, st] + [wspec(a) for a in w2],
        out_specs=[row, row, st, st],
        out_shape=[nd, nd, std, std],
    )(x1, s1, h, stx, sts, *w2)

    w3 = (v2(p['bn2_g']), v2(p['bn2_b']), v2(p['bn2h_g']), v2(p['bn2h_b']))
    x, s2 = pl.pallas_call(
        functools.partial(_post3_body, n),
        grid=(grid,),
        in_specs=[row, row, st, st] + [wspec(a) for a in w3],
        out_specs=[row, row],
        out_shape=[nd, nd],
    )(x2, s2m, stx2, sts2, *w3)
    return x, s2


def kernel(h, state_vectors, edge_index, params):
    n = h.shape[0]
    e = edge_index.shape[1]

    npad = -(-n // ROW_PAD) * ROW_PAD

    kv0, q0, kv1, q1, kv2, q2, kv3, q3 = _projections(h, state_vectors, params)
    accs = _make_sc_edge_kernel(npad, e)(
        kv0, q0, kv1, q1, kv2, q2, kv3, q3, edge_index)
    x, s2 = _post(accs, h, state_vectors, params)
    return x, s2


# speedup vs baseline: 7.3748x; 1.7090x over previous
import functools

import jax
import jax.numpy as jnp
from jax import lax
from jax.experimental import pallas as pl
from jax.experimental.pallas import tpu as pltpu
from jax.experimental.pallas import tpu_sc as plsc

D = 128
H = 8
DH = 16
LANES = 16
NUM_CORES = 2
NUM_SUBCORES = 16

CHUNK = 40

_TAKE_DNUMS = jax.lax.GatherDimensionNumbers(
    offset_dims=(), collapsed_slice_dims=(0,), start_index_map=(0,))
ACCW = 144
ROW_PAD = NUM_SUBCORES * 128


def _proj_body(h_ref, sv_ref, wq_sv, wk_sv, wv_sv, wq_cv, wk_cv, wv_cv,
               wq_sh, wk_sh, wv_sh,
               kv0, q0, kv1, q1, kv2, q2, kv3, q3):
    hb = h_ref[...]
    svb = sv_ref[...]

    def mm(a, w):
        return jnp.dot(a, w[...], preferred_element_type=jnp.float32)

    kv0[:, :D] = mm(hb, wk_sv)
    kv0[:, D:] = mm(hb, wv_sv)
    q0[...] = mm(hb, wq_sv) * 0.25
    kv1[:, :D] = mm(svb, wk_cv)
    kv1[:, D:] = mm(svb, wv_cv)
    q1[...] = mm(hb, wq_cv) * 0.25
    kv2[:, :D] = mm(svb, wk_sh)
    kv2[:, D:] = mm(svb, wv_sh)
    q2[...] = mm(svb, wq_sh) * 0.25
    kv3[:, :D] = mm(hb, wk_cv)
    kv3[:, D:] = mm(hb, wv_cv)
    q3[...] = mm(svb, wq_cv) * 0.25


def _projections(h, sv, p):
    n = h.shape[0]
    blk = 2000
    grid = n // blk
    row_spec = pl.BlockSpec((blk, D), lambda i: (i, 0))
    w_spec = pl.BlockSpec((D, D), lambda i: (0, 0))
    kv_spec = pl.BlockSpec((blk, 2 * D), lambda i: (i, 0))
    return pl.pallas_call(
        _proj_body,
        grid=(grid,),
        in_specs=[row_spec, row_spec] + [w_spec] * 9,
        out_specs=[kv_spec, row_spec] * 4,
        out_shape=[
            s for _ in range(4)
            for s in (jax.ShapeDtypeStruct((n, 2 * D), jnp.float32),
                      jax.ShapeDtypeStruct((n, D), jnp.float32))
        ],
    )(h, sv, p['Wq_sv'], p['Wk_sv'], p['Wv_sv'], p['Wq_cv'], p['Wk_cv'],
      p['Wv_cv'], p['Wq_sh'], p['Wk_sh'], p['Wv_sh'])


def _make_sc_edge_kernel(npad, e):
    ept = e // NUM_SUBCORES
    n_chunks = ept // CHUNK
    rows_per_tile = npad // NUM_SUBCORES
    nz = rows_per_tile // CHUNK

    mesh = plsc.VectorSubcoreMesh(core_axis_name="c", subcore_axis_name="s",
                                  num_cores=NUM_CORES,
                                  num_subcores=NUM_SUBCORES)

    @functools.partial(
        pl.kernel,
        mesh=mesh,
        compiler_params=pltpu.CompilerParams(needs_layout_passes=False,
                                             use_tc_tiling_on_sc=False),
        out_type=[jax.ShapeDtypeStruct((npad, ACCW), jnp.float32)] * 4,
        scratch_types=[
            pltpu.VMEM_SHARED((npad, ACCW), jnp.float32),
            pltpu.VMEM((2, 2, CHUNK), jnp.int32),
            pltpu.VMEM((2, CHUNK, 2 * D), jnp.float32),
            pltpu.VMEM((2, CHUNK, D), jnp.float32),
            pltpu.VMEM((CHUNK, ACCW), jnp.float32),
            pltpu.SemaphoreType.DMA((2,)),
            pltpu.SemaphoreType.DMA((2,)),
        ],
    )
    def sc_kernel(kv0, q0, kv1, q1, kv2, q2, kv3, q3, edge_hbm,
                  out0, out1, out2, out3,
                  acc, sdb, kvb, qb, msgb, sem_kv, sem_q):
        cid = lax.axis_index("c")
        sid = lax.axis_index("s")
        zero16 = jnp.zeros((LANES,), jnp.float32)
        iot = lax.broadcasted_iota(jnp.int32, (LANES,), 0)

        def run_attention(kv_hbm, q_hbm, out_hbm):
            def zero_row(r, carry):
                for kk in range(ACCW // LANES):
                    msgb[r, pl.ds(kk * LANES, LANES)] = zero16
                return carry

            lax.fori_loop(0, CHUNK, zero_row, 0)
            for z in range(nz):
                pltpu.sync_copy(
                    msgb,
                    acc.at[pl.ds(sid * rows_per_tile + z * CHUNK, CHUNK)])
            plsc.subcore_barrier()

            def stage_chunk(i, b):
                pltpu.sync_copy(
                    edge_hbm.at[:, pl.ds(sid * ept + i * CHUNK, CHUNK)],
                    sdb.at[b])
                pltpu.async_copy(kv_hbm.at[sdb.at[b, 0]], kvb.at[b],
                                 sem_kv.at[b])
                pltpu.async_copy(q_hbm.at[sdb.at[b, 1]], qb.at[b],
                                 sem_q.at[b])

            def gathers_wait(b):
                pltpu.make_async_copy(
                    kv_hbm.at[sdb.at[b, 0]], kvb.at[b], sem_kv.at[b]).wait()
                pltpu.make_async_copy(
                    q_hbm.at[sdb.at[b, 1]], qb.at[b], sem_q.at[b]).wait()

            stage_chunk(0, 0)

            def compute_scatter(p):
                kvb_p = kvb.at[p]
                qb_p = qb.at[p]

                @plsc.parallel_loop(0, CHUNK, unroll=1)
                def edge_body(ei):
                    sall = zero16
                    for hh in range(H):
                        kvec = kvb_p[ei, pl.ds(hh * DH, DH)]
                        qvec = qb_p[ei, pl.ds(hh * DH, DH)]
                        dsc = jnp.sum(kvec * qvec)
                        sall = jnp.where(iot == hh, dsc, sall)
                    sall = jnp.exp(jnp.minimum(jnp.maximum(sall, -5.0), 5.0))
                    msgb[ei, pl.ds(D, LANES)] = sall
                    for hh in range(H):
                        hidx = jnp.full((LANES, 1), hh, jnp.int32)
                        svec = lax.gather(
                            sall, hidx, _TAKE_DNUMS, (1,),
                            mode=lax.GatherScatterMode.PROMISE_IN_BOUNDS)
                        vvec = kvb_p[ei, pl.ds(D + hh * DH, DH)]
                        msgb[ei, pl.ds(hh * DH, DH)] = svec * vvec

                pltpu.sync_copy(msgb, acc.at[sdb.at[p, 1]], add=True)

            def chunk_body(i, carry):
                b = lax.rem(i, 2)

                @pl.when(i + 1 < n_chunks)
                def _():
                    stage_chunk(i + 1, 1 - b)

                gathers_wait(b)

                @pl.when(b == 0)
                def _():
                    compute_scatter(0)

                @pl.when(b == 1)
                def _():
                    compute_scatter(1)

                return carry

            lax.fori_loop(0, n_chunks, chunk_body, 0)
            plsc.subcore_barrier()
            pltpu.sync_copy(
                acc.at[pl.ds(sid * rows_per_tile, rows_per_tile)],
                out_hbm.at[pl.ds(sid * rows_per_tile, rows_per_tile)])
            plsc.subcore_barrier()

        @pl.when(cid == 0)
        def _():
            run_attention(kv0, q0, out0)
            run_attention(kv1, q1, out1)

        @pl.when(cid == 1)
        def _():
            run_attention(kv2, q2, out2)
            run_attention(kv3, q3, out3)

    return sc_kernel


def _mm(a, w):
    return jnp.dot(a, w, preferred_element_type=jnp.float32)


def _accum_stats(ref, x):
    st = jnp.concatenate([jnp.sum(x, axis=0, keepdims=True),
                          jnp.sum(x * x, axis=0, keepdims=True)], axis=0)

    @pl.when(pl.program_id(0) == 0)
    def _():
        ref[...] = st

    @pl.when(pl.program_id(0) != 0)
    def _():
        ref[...] += st


def _bn_apply(x, st_ref, n, g, b):
    st = st_ref[...]
    m = st[0:1, :] * (1.0 / n)
    v = st[1:2, :] * (1.0 / n) - m * m
    return g[...] * (x - m) * lax.rsqrt(v + 1e-5) + b[...]


def _post1_body(acc0, acc1, acc2, acc3, h_ref, sv_ref,
                o_w, o_b, oh_w, oh_b, g1_w, g1_b,
                x1_out, s1_out, stx_ref, sts_ref):
    f32 = jnp.float32
    rowi = lax.broadcasted_iota(jnp.int32, (H, D), 0)
    coli = lax.broadcasted_iota(jnp.int32, (H, D), 1)
    expand = (coli // DH == rowi).astype(f32)

    def att(acc_ref):
        a = acc_ref[...]
        return a[:, :D] / _mm(a[:, D:D + H], expand)

    hb = h_ref[...]
    svb = sv_ref[...]
    ow = o_w[...]
    x1 = _mm(att(acc0), ow[:D, :]) + _mm(att(acc1), ow[D:, :]) + o_b[...]
    x1 = hb + x1
    x1_out[...] = x1
    _accum_stats(stx_ref, x1)

    ohw = oh_w[...]
    s1 = _mm(att(acc2), ohw[:D, :]) + _mm(att(acc3), ohw[D:, :]) + oh_b[...]
    g1 = jax.nn.sigmoid(_mm(hb, g1_w[...]) + g1_b[...])
    s1 = (1.0 - g1) * svb + g1 * s1
    s1_out[...] = s1
    _accum_stats(sts_ref, s1)


def _post2_body(n, x1_ref, s1_ref, h_ref, stx_ref, sts_ref,
                f1_w, f1_b, f2_w, f2_b, bn1_g, bn1_b,
                f1h_w, f1h_b, f2h_w, f2h_b, bn1h_g, bn1h_b,
                g2_w, g2_b,
                x2_out, s2_out, stx2_ref, sts2_ref):
    def relu(x):
        return jnp.maximum(x, 0.0)

    xb1 = _bn_apply(x1_ref[...], stx_ref, n, bn1_g, bn1_b)
    y = _mm(relu(_mm(xb1, f1_w[...]) + f1_b[...]), f2_w[...]) + f2_b[...]
    x2 = xb1 + y
    x2_out[...] = x2
    _accum_stats(stx2_ref, x2)

    sb1 = _bn_apply(s1_ref[...], sts_ref, n, bn1h_g, bn1h_b)
    y2 = _mm(relu(_mm(sb1, f1h_w[...]) + f1h_b[...]), f2h_w[...]) + f2h_b[...]
    g2 = jax.nn.sigmoid(_mm(h_ref[...], g2_w[...]) + g2_b[...])
    s2 = (1.0 - g2) * sb1 + g2 * y2
    s2_out[...] = s2
    _accum_stats(sts2_ref, s2)


def _post3_body(n, x2_ref, s2_ref, stx2_ref, sts2_ref,
                bn2_g, bn2_b, bn2h_g, bn2h_b, x_out, s_out):
    x_out[...] = _bn_apply(x2_ref[...], stx2_ref, n, bn2_g, bn2_b)
    s_out[...] = _bn_apply(s2_ref[...], sts2_ref, n, bn2h_g, bn2h_b)


def _post(accs, h, sv, p):
    n = h.shape[0]
    blk = 2000
    grid = n // blk

    def v2(a):
        return a.reshape(1, -1)

    row = pl.BlockSpec((blk, D), lambda i: (i, 0))
    accs_spec = pl.BlockSpec((blk, ACCW), lambda i: (i, 0))
    st = pl.BlockSpec((2, D), lambda i: (0, 0))

    def wspec(a):
        return pl.BlockSpec(a.shape, lambda i: (0,) * a.ndim)

    nd = jax.ShapeDtypeStruct((n, D), jnp.float32)
    std = jax.ShapeDtypeStruct((2, D), jnp.float32)

    w1 = (p['O_w'], v2(p['O_b']), p['Oh_w'], v2(p['Oh_b']),
          p['G1_w'], v2(p['G1_b']))
    x1, s1, stx, sts = pl.pallas_call(
        _post1_body,
        grid=(grid,),
        in_specs=[accs_spec] * 4 + [row, row] + [wspec(a) for a in w1],
        out_specs=[row, row, st, st],
        out_shape=[nd, nd, std, std],
    )(*accs, h, sv, *w1)

    w2 = (p['F1_w'], v2(p['F1_b']), p['F2_w'], v2(p['F2_b']),
          v2(p['bn1_g']), v2(p['bn1_b']),
          p['F1h_w'], v2(p['F1h_b']), p['F2h_w'], v2(p['F2h_b']),
          v2(p['bn1h_g']), v2(p['bn1h_b']), p['G2_w'], v2(p['G2_b']))
    x2, s2m, stx2, sts2 = pl.pallas_call(
        functools.partial(_post2_body, n),
        grid=(grid,),
        in_specs=[row, row, row, st, st] + [wspec(a) for a in w2],
        out_specs=[row, row, st, st],
        out_shape=[nd, nd, std, std],
    )(x1, s1, h, stx, sts, *w2)

    w3 = (v2(p['bn2_g']), v2(p['bn2_b']), v2(p['bn2h_g']), v2(p['bn2h_b']))
    x, s2 = pl.pallas_call(
        functools.partial(_post3_body, n),
        grid=(grid,),
        in_specs=[row, row, st, st] + [wspec(a) for a in w3],
        out_specs=[row, row],
        out_shape=[nd, nd],
    )(x2, s2m, stx2, sts2, *w3)
    return x, s2


def kernel(h, state_vectors, edge_index, params):
    n = h.shape[0]
    e = edge_index.shape[1]

    npad = -(-n // ROW_PAD) * ROW_PAD

    kv0, q0, kv1, q1, kv2, q2, kv3, q3 = _projections(h, state_vectors, params)
    accs = _make_sc_edge_kernel(npad, e)(
        kv0, q0, kv1, q1, kv2, q2, kv3, q3, edge_index)
    x, s2 = _post(accs, h, state_vectors, params)
    return x, s2


# speedup vs baseline: 7.5421x; 1.0227x over previous
import functools

import jax
import jax.numpy as jnp
from jax import lax
from jax.experimental import pallas as pl
from jax.experimental.pallas import tpu as pltpu
from jax.experimental.pallas import tpu_sc as plsc

D = 128
H = 8
DH = 16
LANES = 16
NUM_CORES = 2
NUM_SUBCORES = 16

CHUNK = 32

_TAKE_DNUMS = jax.lax.GatherDimensionNumbers(
    offset_dims=(), collapsed_slice_dims=(0,), start_index_map=(0,))
ACCW = 144
ROW_PAD = NUM_SUBCORES * 128


def _proj_body(h_ref, sv_ref, wq_sv, wk_sv, wv_sv, wq_cv, wk_cv, wv_cv,
               wq_sh, wk_sh, wv_sh,
               kv0, q0, kv1, q1, kv2, q2, kv3, q3):
    hb = h_ref[...]
    svb = sv_ref[...]

    def mm(a, w):
        return jnp.dot(a, w[...], preferred_element_type=jnp.float32)

    kv0[:, :D] = mm(hb, wk_sv)
    kv0[:, D:] = mm(hb, wv_sv)
    q0[...] = mm(hb, wq_sv) * 0.25
    kv1[:, :D] = mm(svb, wk_cv)
    kv1[:, D:] = mm(svb, wv_cv)
    q1[...] = mm(hb, wq_cv) * 0.25
    kv2[:, :D] = mm(svb, wk_sh)
    kv2[:, D:] = mm(svb, wv_sh)
    q2[...] = mm(svb, wq_sh) * 0.25
    kv3[:, :D] = mm(hb, wk_cv)
    kv3[:, D:] = mm(hb, wv_cv)
    q3[...] = mm(svb, wq_cv) * 0.25


def _projections(h, sv, p):
    n = h.shape[0]
    blk = 2000
    grid = n // blk
    row_spec = pl.BlockSpec((blk, D), lambda i: (i, 0))
    w_spec = pl.BlockSpec((D, D), lambda i: (0, 0))
    kv_spec = pl.BlockSpec((blk, 2 * D), lambda i: (i, 0))
    return pl.pallas_call(
        _proj_body,
        grid=(grid,),
        in_specs=[row_spec, row_spec] + [w_spec] * 9,
        out_specs=[kv_spec, row_spec] * 4,
        out_shape=[
            s for _ in range(4)
            for s in (jax.ShapeDtypeStruct((n, 2 * D), jnp.float32),
                      jax.ShapeDtypeStruct((n, D), jnp.float32))
        ],
    )(h, sv, p['Wq_sv'], p['Wk_sv'], p['Wv_sv'], p['Wq_cv'], p['Wk_cv'],
      p['Wv_cv'], p['Wq_sh'], p['Wk_sh'], p['Wv_sh'])


def _make_sc_edge_kernel(npad, e):
    ept = e // NUM_SUBCORES
    n_chunks = ept // CHUNK
    rows_per_tile = npad // NUM_SUBCORES
    nz = rows_per_tile // CHUNK

    mesh = plsc.VectorSubcoreMesh(core_axis_name="c", subcore_axis_name="s",
                                  num_cores=NUM_CORES,
                                  num_subcores=NUM_SUBCORES)

    @functools.partial(
        pl.kernel,
        mesh=mesh,
        compiler_params=pltpu.CompilerParams(needs_layout_passes=False,
                                             use_tc_tiling_on_sc=False),
        out_type=[jax.ShapeDtypeStruct((npad, ACCW), jnp.float32)] * 4,
        scratch_types=[
            pltpu.VMEM_SHARED((npad, ACCW), jnp.float32),
            pltpu.VMEM((4, 2, CHUNK), jnp.int32),
            pltpu.VMEM((2, CHUNK, 2 * D), jnp.float32),
            pltpu.VMEM((2, CHUNK, D), jnp.float32),
            pltpu.VMEM((2, CHUNK, ACCW), jnp.float32),
            pltpu.SemaphoreType.DMA((2,)),
            pltpu.SemaphoreType.DMA((2,)),
            pltpu.SemaphoreType.DMA((2,)),
        ],
    )
    def sc_kernel(kv0, q0, kv1, q1, kv2, q2, kv3, q3, edge_hbm,
                  out0, out1, out2, out3,
                  acc, sdb, kvb, qb, msgb, sem_kv, sem_q, sem_msg):
        cid = lax.axis_index("c")
        sid = lax.axis_index("s")
        zero16 = jnp.zeros((LANES,), jnp.float32)
        iot = lax.broadcasted_iota(jnp.int32, (LANES,), 0)

        def run_attention(kv_hbm, q_hbm, out_hbm):
            def zero_row(r, carry):
                for kk in range(ACCW // LANES):
                    msgb[0, r, pl.ds(kk * LANES, LANES)] = zero16
                return carry

            lax.fori_loop(0, CHUNK, zero_row, 0)
            for z in range(nz):
                pltpu.sync_copy(
                    msgb.at[0],
                    acc.at[pl.ds(sid * rows_per_tile + z * CHUNK, CHUNK)])
            plsc.subcore_barrier()

            def stage_chunk(i, r, b):
                pltpu.sync_copy(
                    edge_hbm.at[:, pl.ds(sid * ept + i * CHUNK, CHUNK)],
                    sdb.at[r])
                pltpu.async_copy(kv_hbm.at[sdb.at[r, 0]], kvb.at[b],
                                 sem_kv.at[b])
                pltpu.async_copy(q_hbm.at[sdb.at[r, 1]], qb.at[b],
                                 sem_q.at[b])

            def gathers_wait(r, b):
                pltpu.make_async_copy(
                    kv_hbm.at[sdb.at[r, 0]], kvb.at[b], sem_kv.at[b]).wait()
                pltpu.make_async_copy(
                    q_hbm.at[sdb.at[r, 1]], qb.at[b], sem_q.at[b]).wait()

            def scatter_wait(r, b):
                pltpu.make_async_copy(
                    msgb.at[b], acc.at[sdb.at[r, 1]], sem_msg.at[b]).wait()

            stage_chunk(0, 0, 0)

            def compute_scatter(r, p):
                kvb_p = kvb.at[p]
                qb_p = qb.at[p]

                @plsc.parallel_loop(0, CHUNK, unroll=1)
                def edge_body(ei):
                    sall = zero16
                    for hh in range(H):
                        kvec = kvb_p[ei, pl.ds(hh * DH, DH)]
                        qvec = qb_p[ei, pl.ds(hh * DH, DH)]
                        dsc = jnp.sum(kvec * qvec)
                        sall = jnp.where(iot == hh, dsc, sall)
                    sall = jnp.exp(jnp.minimum(jnp.maximum(sall, -5.0), 5.0))
                    msgb[p, ei, pl.ds(D, LANES)] = sall
                    for hh in range(H):
                        hidx = jnp.full((LANES, 1), hh, jnp.int32)
                        svec = lax.gather(
                            sall, hidx, _TAKE_DNUMS, (1,),
                            mode=lax.GatherScatterMode.PROMISE_IN_BOUNDS)
                        vvec = kvb_p[ei, pl.ds(D + hh * DH, DH)]
                        msgb[p, ei, pl.ds(hh * DH, DH)] = svec * vvec

                pltpu.async_copy(msgb.at[p], acc.at[sdb.at[r, 1]],
                                 sem_msg.at[p], add=True)

            def chunk_body(i, carry):
                m = lax.rem(i, 4)
                for mi in range(4):
                    b = mi % 2

                    @pl.when(m == mi)
                    def _(mi=mi, b=b):
                        @pl.when(i + 1 < n_chunks)
                        def _():
                            stage_chunk(i + 1, (mi + 1) % 4, 1 - b)

                        gathers_wait(mi, b)

                        @pl.when(i >= 2)
                        def _():
                            scatter_wait((mi + 2) % 4, b)

                        compute_scatter(mi, b)

                return carry

            lax.fori_loop(0, n_chunks, chunk_body, 0)
            for j in (n_chunks - 2, n_chunks - 1):
                if j >= 0:
                    scatter_wait(j % 4, j % 2)
            plsc.subcore_barrier()
            pltpu.sync_copy(
                acc.at[pl.ds(sid * rows_per_tile, rows_per_tile)],
                out_hbm.at[pl.ds(sid * rows_per_tile, rows_per_tile)])
            plsc.subcore_barrier()

        @pl.when(cid == 0)
        def _():
            run_attention(kv0, q0, out0)
            run_attention(kv1, q1, out1)

        @pl.when(cid == 1)
        def _():
            run_attention(kv2, q2, out2)
            run_attention(kv3, q3, out3)

    return sc_kernel


def _mm(a, w):
    return jnp.dot(a, w, preferred_element_type=jnp.float32)


def _accum_stats(ref, x):
    st = jnp.concatenate([jnp.sum(x, axis=0, keepdims=True),
                          jnp.sum(x * x, axis=0, keepdims=True)], axis=0)

    @pl.when(pl.program_id(0) == 0)
    def _():
        ref[...] = st

    @pl.when(pl.program_id(0) != 0)
    def _():
        ref[...] += st


def _bn_apply(x, st_ref, n, g, b):
    st = st_ref[...]
    m = st[0:1, :] * (1.0 / n)
    v = st[1:2, :] * (1.0 / n) - m * m
    return g[...] * (x - m) * lax.rsqrt(v + 1e-5) + b[...]


def _post1_body(acc0, acc1, acc2, acc3, h_ref, sv_ref,
                o_w, o_b, oh_w, oh_b, g1_w, g1_b,
                x1_out, s1_out, stx_ref, sts_ref):
    f32 = jnp.float32
    rowi = lax.broadcasted_iota(jnp.int32, (H, D), 0)
    coli = lax.broadcasted_iota(jnp.int32, (H, D), 1)
    expand = (coli // DH == rowi).astype(f32)

    def att(acc_ref):
        a = acc_ref[...]
        return a[:, :D] / _mm(a[:, D:D + H], expand)

    hb = h_ref[...]
    svb = sv_ref[...]
    ow = o_w[...]
    x1 = _mm(att(acc0), ow[:D, :]) + _mm(att(acc1), ow[D:, :]) + o_b[...]
    x1 = hb + x1
    x1_out[...] = x1
    _accum_stats(stx_ref, x1)

    ohw = oh_w[...]
    s1 = _mm(att(acc2), ohw[:D, :]) + _mm(att(acc3), ohw[D:, :]) + oh_b[...]
    g1 = jax.nn.sigmoid(_mm(hb, g1_w[...]) + g1_b[...])
    s1 = (1.0 - g1) * svb + g1 * s1
    s1_out[...] = s1
    _accum_stats(sts_ref, s1)


def _post2_body(n, x1_ref, s1_ref, h_ref, stx_ref, sts_ref,
                f1_w, f1_b, f2_w, f2_b, bn1_g, bn1_b,
                f1h_w, f1h_b, f2h_w, f2h_b, bn1h_g, bn1h_b,
                g2_w, g2_b,
                x2_out, s2_out, stx2_ref, sts2_ref):
    def relu(x):
        return jnp.maximum(x, 0.0)

    xb1 = _bn_apply(x1_ref[...], stx_ref, n, bn1_g, bn1_b)
    y = _mm(relu(_mm(xb1, f1_w[...]) + f1_b[...]), f2_w[...]) + f2_b[...]
    x2 = xb1 + y
    x2_out[...] = x2
    _accum_stats(stx2_ref, x2)

    sb1 = _bn_apply(s1_ref[...], sts_ref, n, bn1h_g, bn1h_b)
    y2 = _mm(relu(_mm(sb1, f1h_w[...]) + f1h_b[...]), f2h_w[...]) + f2h_b[...]
    g2 = jax.nn.sigmoid(_mm(h_ref[...], g2_w[...]) + g2_b[...])
    s2 = (1.0 - g2) * sb1 + g2 * y2
    s2_out[...] = s2
    _accum_stats(sts2_ref, s2)


def _post3_body(n, x2_ref, s2_ref, stx2_ref, sts2_ref,
                bn2_g, bn2_b, bn2h_g, bn2h_b, x_out, s_out):
    x_out[...] = _bn_apply(x2_ref[...], stx2_ref, n, bn2_g, bn2_b)
    s_out[...] = _bn_apply(s2_ref[...], sts2_ref, n, bn2h_g, bn2h_b)


def _post(accs, h, sv, p):
    n = h.shape[0]
    blk = 2000
    grid = n // blk

    def v2(a):
        return a.reshape(1, -1)

    row = pl.BlockSpec((blk, D), lambda i: (i, 0))
    accs_spec = pl.BlockSpec((blk, ACCW), lambda i: (i, 0))
    st = pl.BlockSpec((2, D), lambda i: (0, 0))

    def wspec(a):
        return pl.BlockSpec(a.shape, lambda i: (0,) * a.ndim)

    nd = jax.ShapeDtypeStruct((n, D), jnp.float32)
    std = jax.ShapeDtypeStruct((2, D), jnp.float32)

    w1 = (p['O_w'], v2(p['O_b']), p['Oh_w'], v2(p['Oh_b']),
          p['G1_w'], v2(p['G1_b']))
    x1, s1, stx, sts = pl.pallas_call(
        _post1_body,
        grid=(grid,),
        in_specs=[accs_spec] * 4 + [row, row] + [wspec(a) for a in w1],
        out_specs=[row, row, st, st],
        out_shape=[nd, nd, std, std],
    )(*accs, h, sv, *w1)

    w2 = (p['F1_w'], v2(p['F1_b']), p['F2_w'], v2(p['F2_b']),
          v2(p['bn1_g']), v2(p['bn1_b']),
          p['F1h_w'], v2(p['F1h_b']), p['F2h_w'], v2(p['F2h_b']),
          v2(p['bn1h_g']), v2(p['bn1h_b']), p['G2_w'], v2(p['G2_b']))
    x2, s2m, stx2, sts2 = pl.pallas_call(
        functools.partial(_post2_body, n),
        grid=(grid,),
        in_specs=[row, row, row, st, st] + [wspec(a) for a in w2],
        out_specs=[row, row, st, st],
        out_shape=[nd, nd, std, std],
    )(x1, s1, h, stx, sts, *w2)

    w3 = (v2(p['bn2_g']), v2(p['bn2_b']), v2(p['bn2h_g']), v2(p['bn2h_b']))
    x, s2 = pl.pallas_call(
        functools.partial(_post3_body, n),
        grid=(grid,),
        in_specs=[row, row, st, st] + [wspec(a) for a in w3],
        out_specs=[row, row],
        out_shape=[nd, nd],
    )(x2, s2m, stx2, sts2, *w3)
    return x, s2


def kernel(h, state_vectors, edge_index, params):
    n = h.shape[0]
    e = edge_index.shape[1]

    npad = -(-n // ROW_PAD) * ROW_PAD

    kv0, q0, kv1, q1, kv2, q2, kv3, q3 = _projections(h, state_vectors, params)
    accs = _make_sc_edge_kernel(npad, e)(
        kv0, q0, kv1, q1, kv2, q2, kv3, q3, edge_index)
    x, s2 = _post(accs, h, state_vectors, params)
    return x, s2


# speedup vs baseline: 7.5427x; 1.0001x over previous
import functools

import jax
import jax.numpy as jnp
from jax import lax
from jax.experimental import pallas as pl
from jax.experimental.pallas import tpu as pltpu
from jax.experimental.pallas import tpu_sc as plsc

D = 128
H = 8
DH = 16
LANES = 16
NUM_CORES = 2
NUM_SUBCORES = 16

CHUNK = 32

_TAKE_DNUMS = jax.lax.GatherDimensionNumbers(
    offset_dims=(), collapsed_slice_dims=(0,), start_index_map=(0,))
ACCW = 144
ROW_PAD = NUM_SUBCORES * 128


def _proj_body(h_ref, sv_ref, wq_sv, wk_sv, wv_sv, wq_cv, wk_cv, wv_cv,
               wq_sh, wk_sh, wv_sh,
               kv0, q0, kv1, q1, kv2, q2, kv3, q3):
    hb = h_ref[...]
    svb = sv_ref[...]

    def mm(a, w):
        return jnp.dot(a, w[...], preferred_element_type=jnp.float32)

    kv0[:, :D] = mm(hb, wk_sv)
    kv0[:, D:] = mm(hb, wv_sv)
    q0[...] = mm(hb, wq_sv) * 0.25
    kv1[:, :D] = mm(svb, wk_cv)
    kv1[:, D:] = mm(svb, wv_cv)
    q1[...] = mm(hb, wq_cv) * 0.25
    kv2[:, :D] = mm(svb, wk_sh)
    kv2[:, D:] = mm(svb, wv_sh)
    q2[...] = mm(svb, wq_sh) * 0.25
    kv3[:, :D] = mm(hb, wk_cv)
    kv3[:, D:] = mm(hb, wv_cv)
    q3[...] = mm(svb, wq_cv) * 0.25


def _projections(h, sv, p):
    n = h.shape[0]
    blk = 2000
    grid = n // blk
    row_spec = pl.BlockSpec((blk, D), lambda i: (i, 0))
    w_spec = pl.BlockSpec((D, D), lambda i: (0, 0))
    kv_spec = pl.BlockSpec((blk, 2 * D), lambda i: (i, 0))
    return pl.pallas_call(
        _proj_body,
        grid=(grid,),
        in_specs=[row_spec, row_spec] + [w_spec] * 9,
        out_specs=[kv_spec, row_spec] * 4,
        out_shape=[
            s for _ in range(4)
            for s in (jax.ShapeDtypeStruct((n, 2 * D), jnp.float32),
                      jax.ShapeDtypeStruct((n, D), jnp.float32))
        ],
    )(h, sv, p['Wq_sv'], p['Wk_sv'], p['Wv_sv'], p['Wq_cv'], p['Wk_cv'],
      p['Wv_cv'], p['Wq_sh'], p['Wk_sh'], p['Wv_sh'])


def _make_sc_edge_kernel(npad, e):
    ept = e // NUM_SUBCORES
    n_chunks = ept // CHUNK
    rows_per_tile = npad // NUM_SUBCORES
    nz = rows_per_tile // CHUNK

    mesh = plsc.VectorSubcoreMesh(core_axis_name="c", subcore_axis_name="s",
                                  num_cores=NUM_CORES,
                                  num_subcores=NUM_SUBCORES)

    @functools.partial(
        pl.kernel,
        mesh=mesh,
        compiler_params=pltpu.CompilerParams(needs_layout_passes=False,
                                             use_tc_tiling_on_sc=False),
        out_type=[jax.ShapeDtypeStruct((npad, ACCW), jnp.float32)] * 4,
        scratch_types=[
            pltpu.VMEM_SHARED((npad, ACCW), jnp.float32),
            pltpu.VMEM((4, 2, CHUNK), jnp.int32),
            pltpu.VMEM((2, CHUNK, 2 * D), jnp.float32),
            pltpu.VMEM((2, CHUNK, D), jnp.float32),
            pltpu.VMEM((2, CHUNK, ACCW), jnp.float32),
            pltpu.SemaphoreType.DMA((2,)),
            pltpu.SemaphoreType.DMA((2,)),
            pltpu.SemaphoreType.DMA((2,)),
        ],
    )
    def sc_kernel(kv0, q0, kv1, q1, kv2, q2, kv3, q3, edge_hbm,
                  out0, out1, out2, out3,
                  acc, sdb, kvb, qb, msgb, sem_kv, sem_q, sem_msg):
        cid = lax.axis_index("c")
        sid = lax.axis_index("s")
        zero16 = jnp.zeros((LANES,), jnp.float32)
        iot = lax.broadcasted_iota(jnp.int32, (LANES,), 0)

        def run_attention(kv_hbm, q_hbm, out_hbm):
            def zero_row(r, carry):
                for kk in range(ACCW // LANES):
                    msgb[0, r, pl.ds(kk * LANES, LANES)] = zero16
                return carry

            lax.fori_loop(0, CHUNK, zero_row, 0)
            for z in range(nz):
                pltpu.sync_copy(
                    msgb.at[0],
                    acc.at[pl.ds(sid * rows_per_tile + z * CHUNK, CHUNK)])
            plsc.subcore_barrier()

            def stage_chunk(i, r, b):
                pltpu.sync_copy(
                    edge_hbm.at[:, pl.ds(sid * ept + i * CHUNK, CHUNK)],
                    sdb.at[r])
                pltpu.async_copy(kv_hbm.at[sdb.at[r, 0]], kvb.at[b],
                                 sem_kv.at[b])
                pltpu.async_copy(q_hbm.at[sdb.at[r, 1]], qb.at[b],
                                 sem_q.at[b])

            def gathers_wait(r, b):
                pltpu.make_async_copy(
                    kv_hbm.at[sdb.at[r, 0]], kvb.at[b], sem_kv.at[b]).wait()
                pltpu.make_async_copy(
                    q_hbm.at[sdb.at[r, 1]], qb.at[b], sem_q.at[b]).wait()

            def scatter_wait(r, b):
                pltpu.make_async_copy(
                    msgb.at[b], acc.at[sdb.at[r, 1]], sem_msg.at[b]).wait()

            stage_chunk(0, 0, 0)

            def compute_scatter(r, p):
                kvb_p = kvb.at[p]
                qb_p = qb.at[p]

                @plsc.parallel_loop(0, CHUNK, unroll=1)
                def edge_body(ei):
                    lane15 = jnp.full((LANES, 1), LANES - 1, jnp.int32)
                    sall = zero16
                    for hh in range(H):
                        kvec = kvb_p[ei, pl.ds(hh * DH, DH)]
                        qvec = qb_p[ei, pl.ds(hh * DH, DH)]
                        ps = jnp.cumsum(kvec * qvec)
                        dv = lax.gather(
                            ps, lane15, _TAKE_DNUMS, (1,),
                            mode=lax.GatherScatterMode.PROMISE_IN_BOUNDS)
                        sall = jnp.where(iot == hh, dv, sall)
                    sall = jnp.exp(jnp.minimum(jnp.maximum(sall, -5.0), 5.0))
                    msgb[p, ei, pl.ds(D, LANES)] = sall
                    for hh in range(H):
                        hidx = jnp.full((LANES, 1), hh, jnp.int32)
                        svec = lax.gather(
                            sall, hidx, _TAKE_DNUMS, (1,),
                            mode=lax.GatherScatterMode.PROMISE_IN_BOUNDS)
                        vvec = kvb_p[ei, pl.ds(D + hh * DH, DH)]
                        msgb[p, ei, pl.ds(hh * DH, DH)] = svec * vvec

                pltpu.async_copy(msgb.at[p], acc.at[sdb.at[r, 1]],
                                 sem_msg.at[p], add=True)

            def chunk_body(i, carry):
                m = lax.rem(i, 4)
                for mi in range(4):
                    b = mi % 2

                    @pl.when(m == mi)
                    def _(mi=mi, b=b):
                        @pl.when(i + 1 < n_chunks)
                        def _():
                            stage_chunk(i + 1, (mi + 1) % 4, 1 - b)

                        gathers_wait(mi, b)

                        @pl.when(i >= 2)
                        def _():
                            scatter_wait((mi + 2) % 4, b)

                        compute_scatter(mi, b)

                return carry

            lax.fori_loop(0, n_chunks, chunk_body, 0)
            for j in (n_chunks - 2, n_chunks - 1):
                if j >= 0:
                    scatter_wait(j % 4, j % 2)
            plsc.subcore_barrier()
            pltpu.sync_copy(
                acc.at[pl.ds(sid * rows_per_tile, rows_per_tile)],
                out_hbm.at[pl.ds(sid * rows_per_tile, rows_per_tile)])
            plsc.subcore_barrier()

        @pl.when(cid == 0)
        def _():
            run_attention(kv0, q0, out0)
            run_attention(kv1, q1, out1)

        @pl.when(cid == 1)
        def _():
            run_attention(kv2, q2, out2)
            run_attention(kv3, q3, out3)

    return sc_kernel


def _mm(a, w):
    return jnp.dot(a, w, preferred_element_type=jnp.float32)


def _accum_stats(ref, x):
    st = jnp.concatenate([jnp.sum(x, axis=0, keepdims=True),
                          jnp.sum(x * x, axis=0, keepdims=True)], axis=0)

    @pl.when(pl.program_id(0) == 0)
    def _():
        ref[...] = st

    @pl.when(pl.program_id(0) != 0)
    def _():
        ref[...] += st


def _bn_apply(x, st_ref, n, g, b):
    st = st_ref[...]
    m = st[0:1, :] * (1.0 / n)
    v = st[1:2, :] * (1.0 / n) - m * m
    return g[...] * (x - m) * lax.rsqrt(v + 1e-5) + b[...]


def _post1_body(acc0, acc1, acc2, acc3, h_ref, sv_ref,
                o_w, o_b, oh_w, oh_b, g1_w, g1_b,
                x1_out, s1_out, stx_ref, sts_ref):
    f32 = jnp.float32
    rowi = lax.broadcasted_iota(jnp.int32, (H, D), 0)
    coli = lax.broadcasted_iota(jnp.int32, (H, D), 1)
    expand = (coli // DH == rowi).astype(f32)

    def att(acc_ref):
        a = acc_ref[...]
        return a[:, :D] / _mm(a[:, D:D + H], expand)

    hb = h_ref[...]
    svb = sv_ref[...]
    ow = o_w[...]
    x1 = _mm(att(acc0), ow[:D, :]) + _mm(att(acc1), ow[D:, :]) + o_b[...]
    x1 = hb + x1
    x1_out[...] = x1
    _accum_stats(stx_ref, x1)

    ohw = oh_w[...]
    s1 = _mm(att(acc2), ohw[:D, :]) + _mm(att(acc3), ohw[D:, :]) + oh_b[...]
    g1 = jax.nn.sigmoid(_mm(hb, g1_w[...]) + g1_b[...])
    s1 = (1.0 - g1) * svb + g1 * s1
    s1_out[...] = s1
    _accum_stats(sts_ref, s1)


def _post2_body(n, x1_ref, s1_ref, h_ref, stx_ref, sts_ref,
                f1_w, f1_b, f2_w, f2_b, bn1_g, bn1_b,
                f1h_w, f1h_b, f2h_w, f2h_b, bn1h_g, bn1h_b,
                g2_w, g2_b,
                x2_out, s2_out, stx2_ref, sts2_ref):
    def relu(x):
        return jnp.maximum(x, 0.0)

    xb1 = _bn_apply(x1_ref[...], stx_ref, n, bn1_g, bn1_b)
    y = _mm(relu(_mm(xb1, f1_w[...]) + f1_b[...]), f2_w[...]) + f2_b[...]
    x2 = xb1 + y
    x2_out[...] = x2
    _accum_stats(stx2_ref, x2)

    sb1 = _bn_apply(s1_ref[...], sts_ref, n, bn1h_g, bn1h_b)
    y2 = _mm(relu(_mm(sb1, f1h_w[...]) + f1h_b[...]), f2h_w[...]) + f2h_b[...]
    g2 = jax.nn.sigmoid(_mm(h_ref[...], g2_w[...]) + g2_b[...])
    s2 = (1.0 - g2) * sb1 + g2 * y2
    s2_out[...] = s2
    _accum_stats(sts2_ref, s2)


def _post3_body(n, x2_ref, s2_ref, stx2_ref, sts2_ref,
                bn2_g, bn2_b, bn2h_g, bn2h_b, x_out, s_out):
    x_out[...] = _bn_apply(x2_ref[...], stx2_ref, n, bn2_g, bn2_b)
    s_out[...] = _bn_apply(s2_ref[...], sts2_ref, n, bn2h_g, bn2h_b)


def _post(accs, h, sv, p):
    n = h.shape[0]
    blk = 2000
    grid = n // blk

    def v2(a):
        return a.reshape(1, -1)

    row = pl.BlockSpec((blk, D), lambda i: (i, 0))
    accs_spec = pl.BlockSpec((blk, ACCW), lambda i: (i, 0))
    st = pl.BlockSpec((2, D), lambda i: (0, 0))

    def wspec(a):
        return pl.BlockSpec(a.shape, lambda i: (0,) * a.ndim)

    nd = jax.ShapeDtypeStruct((n, D), jnp.float32)
    std = jax.ShapeDtypeStruct((2, D), jnp.float32)

    w1 = (p['O_w'], v2(p['O_b']), p['Oh_w'], v2(p['Oh_b']),
          p['G1_w'], v2(p['G1_b']))
    x1, s1, stx, sts = pl.pallas_call(
        _post1_body,
        grid=(grid,),
        in_specs=[accs_spec] * 4 + [row, row] + [wspec(a) for a in w1],
        out_specs=[row, row, st, st],
        out_shape=[nd, nd, std, std],
    )(*accs, h, sv, *w1)

    w2 = (p['F1_w'], v2(p['F1_b']), p['F2_w'], v2(p['F2_b']),
          v2(p['bn1_g']), v2(p['bn1_b']),
          p['F1h_w'], v2(p['F1h_b']), p['F2h_w'], v2(p['F2h_b']),
          v2(p['bn1h_g']), v2(p['bn1h_b']), p['G2_w'], v2(p['G2_b']))
    x2, s2m, stx2, sts2 = pl.pallas_call(
        functools.partial(_post2_body, n),
        grid=(grid,),
        in_specs=[row, row, row, st, st] + [wspec(a) for a in w2],
        out_specs=[row, row, st, st],
        out_shape=[nd, nd, std, std],
    )(x1, s1, h, stx, sts, *w2)

    w3 = (v2(p['bn2_g']), v2(p['bn2_b']), v2(p['bn2h_g']), v2(p['bn2h_b']))
    x, s2 = pl.pallas_call(
        functools.partial(_post3_body, n),
        grid=(grid,),
        in_specs=[row, row, st, st] + [wspec(a) for a in w3],
        out_specs=[row, row],
        out_shape=[nd, nd],
    )(x2, s2m, stx2, sts2, *w3)
    return x, s2


def kernel(h, state_vectors, edge_index, params):
    n = h.shape[0]
    e = edge_index.shape[1]

    npad = -(-n // ROW_PAD) * ROW_PAD

    kv0, q0, kv1, q1, kv2, q2, kv3, q3 = _projections(h, state_vectors, params)
    accs = _make_sc_edge_kernel(npad, e)(
        kv0, q0, kv1, q1, kv2, q2, kv3, q3, edge_index)
    x, s2 = _post(accs, h, state_vectors, params)
    return x, s2
